# baseline (speedup 1.0000x reference)
import functools

import jax
import jax.numpy as jnp
from jax import lax
from jax.experimental import pallas as pl
from jax.experimental.pallas import tpu as pltpu
from jax.experimental.pallas import tpu_sc as plsc

N_NODES = 10000
N_EDGES = 320000
D = 128

NC = 2
NS = 16
NW = NC * NS
CHUNK = 128
EDGES_PER_TILE = N_EDGES // NW
NCHUNK = 80
HALF = NCHUNK // 2
NPAIR_H = HALF // 2
ACC_ROWS = 10240
ROW_BLOCK = 1000


def _make_scatter_kernel():
    mesh = plsc.VectorSubcoreMesh(core_axis_name="c", subcore_axis_name="s")

    @functools.partial(
        pl.kernel,
        mesh=mesh,
        out_type=jax.ShapeDtypeStruct((NC, ACC_ROWS, D), jnp.float32),
        scratch_types=[
            pltpu.VMEM((HALF, CHUNK), jnp.int32),
            pltpu.VMEM((HALF, CHUNK), jnp.int32),
            pltpu.VMEM((CHUNK, D), jnp.float32),
            pltpu.VMEM((CHUNK, D), jnp.float32),
            pltpu.VMEM_SHARED((ACC_ROWS, D), jnp.float32),
            pltpu.SemaphoreType.DMA,
            pltpu.SemaphoreType.DMA,
            pltpu.SemaphoreType.DMA,
            pltpu.SemaphoreType.DMA,
        ],
    )
    def scatter_kernel(src_hbm, dst_hbm, x_hbm, out_hbm,
                       src_v, dst_v, rows0, rows1, acc,
                       sem0a, sem0b, sem1a, sem1b):
        cid = lax.axis_index("c")
        sid = lax.axis_index("s")
        wid = sid * NC + cid

        zeros16 = jnp.zeros((16,), jnp.float32)

        def zrow(i, carry):
            for j in range(D // 16):
                rows0[i, pl.ds(j * 16, 16)] = zeros16
            return carry

        lax.fori_loop(0, CHUNK, zrow, 0)
        rows_per_tile = ACC_ROWS // NS

        def zacc(k, carry):
            pltpu.sync_copy(
                rows0, acc.at[pl.ds(sid * rows_per_tile + k * CHUNK, CHUNK)])
            return carry

        lax.fori_loop(0, rows_per_tile // CHUNK, zacc, 0)
        plsc.subcore_barrier()

        HC = CHUNK // 2

        def gather(j, rows, sa, sb):
            pltpu.async_copy(
                x_hbm.at[src_v.at[j, pl.ds(0, HC)]], rows.at[pl.ds(0, HC)], sa)
            pltpu.async_copy(
                x_hbm.at[src_v.at[j, pl.ds(HC, HC)]], rows.at[pl.ds(HC, HC)],
                sb)

        def gwait(j, rows, sa, sb):
            pltpu.make_async_copy(
                x_hbm.at[src_v.at[j, pl.ds(0, HC)]], rows.at[pl.ds(0, HC)],
                sa).wait()
            pltpu.make_async_copy(
                x_hbm.at[src_v.at[j, pl.ds(HC, HC)]], rows.at[pl.ds(HC, HC)],
                sb).wait()

        for h in range(NCHUNK // HALF):
            pltpu.sync_copy(src_hbm.at[wid, pl.ds(h * HALF, HALF)], src_v)
            pltpu.sync_copy(dst_hbm.at[wid, pl.ds(h * HALF, HALF)], dst_v)
            gather(0, rows0, sem0a, sem0b)

            def step(k, carry):
                j0 = 2 * k
                j1 = j0 + 1
                gwait(j0, rows0, sem0a, sem0b)
                gather(j1, rows1, sem1a, sem1b)
                pltpu.sync_copy(rows0, acc.at[dst_v.at[j0]], add=True)

                @pl.when(k < NPAIR_H - 1)
                def _():
                    gather(j0 + 2, rows0, sem0a, sem0b)

                gwait(j1, rows1, sem1a, sem1b)
                pltpu.sync_copy(rows1, acc.at[dst_v.at[j1]], add=True)
                return carry

            lax.fori_loop(0, NPAIR_H, step, 0)
        plsc.subcore_barrier()

        pltpu.sync_copy(
            acc.at[pl.ds(sid * rows_per_tile, rows_per_tile)],
            out_hbm.at[cid, pl.ds(sid * rows_per_tile, rows_per_tile)])

    return scatter_kernel


_scatter = _make_scatter_kernel()


def _combine_body(p_ref, w_ref, b_ref, o_ref):
    agg = p_ref[0] + p_ref[1]
    o_ref[...] = (
        jnp.dot(agg, w_ref[...], preferred_element_type=jnp.float32)
        + b_ref[...])


_combine = pl.pallas_call(
    _combine_body,
    grid=(N_NODES // ROW_BLOCK,),
    in_specs=[
        pl.BlockSpec((NC, ROW_BLOCK, D), lambda i: (0, i, 0)),
        pl.BlockSpec((D, D), lambda i: (0, 0)),
        pl.BlockSpec((1, D), lambda i: (0, 0)),
    ],
    out_specs=pl.BlockSpec((ROW_BLOCK, D), lambda i: (i, 0)),
    out_shape=jax.ShapeDtypeStruct((N_NODES, D), jnp.float32),
)


def kernel(x, edge_index, W, b):
    ei = edge_index.astype(jnp.int32)
    ppt = NCHUNK * CHUNK - EDGES_PER_TILE
    lane = jnp.arange(ppt, dtype=jnp.int32)[None, :]
    tile = jnp.arange(NW, dtype=jnp.int32)[:, None]
    pad_src = (tile * 113 + lane * 89) % N_NODES
    pad_dst = N_NODES + (tile * 7 + lane) % (ACC_ROWS - N_NODES)
    src = jnp.concatenate(
        [ei[0].reshape(NW, EDGES_PER_TILE), pad_src], axis=1
    ).reshape(NW, NCHUNK, CHUNK)
    dst = jnp.concatenate(
        [ei[1].reshape(NW, EDGES_PER_TILE), pad_dst], axis=1
    ).reshape(NW, NCHUNK, CHUNK)
    partials = _scatter(src, dst, x)
    return _combine(partials, W, b.reshape(1, D))


# speedup vs baseline: 1.0293x; 1.0293x over previous
import functools

import jax
import jax.numpy as jnp
from jax import lax
from jax.experimental import pallas as pl
from jax.experimental.pallas import tpu as pltpu
from jax.experimental.pallas import tpu_sc as plsc

N_NODES = 10000
N_EDGES = 320000
D = 128

NC = 2
NS = 16
NW = NC * NS
CHUNK = 128
EDGES_PER_TILE = N_EDGES // NW
NCHUNK = 80
HALF = NCHUNK // 2
NPAIR_H = HALF // 2
ACC_ROWS = 10240
ROW_BLOCK = 2000


def _make_scatter_kernel():
    mesh = plsc.VectorSubcoreMesh(core_axis_name="c", subcore_axis_name="s")

    @functools.partial(
        pl.kernel,
        mesh=mesh,
        out_type=jax.ShapeDtypeStruct((NC, ACC_ROWS, D), jnp.float32),
        scratch_types=[
            pltpu.VMEM((HALF, CHUNK), jnp.int32),
            pltpu.VMEM((HALF, CHUNK), jnp.int32),
            pltpu.VMEM((CHUNK, D), jnp.float32),
            pltpu.VMEM((CHUNK, D), jnp.float32),
            pltpu.VMEM_SHARED((ACC_ROWS, D), jnp.float32),
            pltpu.SemaphoreType.DMA,
            pltpu.SemaphoreType.DMA,
        ],
    )
    def scatter_kernel(src_hbm, dst_hbm, x_hbm, out_hbm,
                       src_v, dst_v, rows0, rows1, acc, sem0a, sem1a):
        cid = lax.axis_index("c")
        sid = lax.axis_index("s")
        wid = sid * NC + cid

        zeros16 = jnp.zeros((16,), jnp.float32)

        def zrow(i, carry):
            for j in range(D // 16):
                rows0[i, pl.ds(j * 16, 16)] = zeros16
            return carry

        lax.fori_loop(0, CHUNK, zrow, 0)
        rows_per_tile = ACC_ROWS // NS

        def zacc(k, carry):
            pltpu.sync_copy(
                rows0, acc.at[pl.ds(sid * rows_per_tile + k * CHUNK, CHUNK)])
            return carry

        lax.fori_loop(0, rows_per_tile // CHUNK, zacc, 0)
        plsc.subcore_barrier()

        for h in range(NCHUNK // HALF):
            pltpu.sync_copy(src_hbm.at[wid, pl.ds(h * HALF, HALF)], src_v)
            pltpu.sync_copy(dst_hbm.at[wid, pl.ds(h * HALF, HALF)], dst_v)
            pltpu.async_copy(x_hbm.at[src_v.at[0]], rows0, sem0a)

            def step(k, carry):
                j0 = 2 * k
                j1 = j0 + 1
                pltpu.make_async_copy(
                    x_hbm.at[src_v.at[j0]], rows0, sem0a).wait()
                pltpu.async_copy(x_hbm.at[src_v.at[j1]], rows1, sem1a)
                pltpu.sync_copy(rows0, acc.at[dst_v.at[j0]], add=True)

                @pl.when(k < NPAIR_H - 1)
                def _():
                    pltpu.async_copy(x_hbm.at[src_v.at[j0 + 2]], rows0, sem0a)

                pltpu.make_async_copy(
                    x_hbm.at[src_v.at[j1]], rows1, sem1a).wait()
                pltpu.sync_copy(rows1, acc.at[dst_v.at[j1]], add=True)
                return carry

            lax.fori_loop(0, NPAIR_H, step, 0)
        plsc.subcore_barrier()

        pltpu.sync_copy(
            acc.at[pl.ds(sid * rows_per_tile, rows_per_tile)],
            out_hbm.at[cid, pl.ds(sid * rows_per_tile, rows_per_tile)])

    return scatter_kernel


_scatter = _make_scatter_kernel()


def _combine_body(p_ref, w_ref, b_ref, o_ref):
    agg = p_ref[0] + p_ref[1]
    o_ref[...] = (
        jnp.dot(agg, w_ref[...], preferred_element_type=jnp.float32)
        + b_ref[...])


_combine = pl.pallas_call(
    _combine_body,
    grid=(N_NODES // ROW_BLOCK,),
    in_specs=[
        pl.BlockSpec((NC, ROW_BLOCK, D), lambda i: (0, i, 0)),
        pl.BlockSpec((D, D), lambda i: (0, 0)),
        pl.BlockSpec((1, D), lambda i: (0, 0)),
    ],
    out_specs=pl.BlockSpec((ROW_BLOCK, D), lambda i: (i, 0)),
    out_shape=jax.ShapeDtypeStruct((N_NODES, D), jnp.float32),
)


def kernel(x, edge_index, W, b):
    ei = edge_index.astype(jnp.int32)
    ppt = NCHUNK * CHUNK - EDGES_PER_TILE
    lane = jnp.arange(ppt, dtype=jnp.int32)[None, :]
    tile = jnp.arange(NW, dtype=jnp.int32)[:, None]
    pad_src = (tile * 113 + lane * 89) % N_NODES
    pad_dst = N_NODES + (tile * 7 + lane) % (ACC_ROWS - N_NODES)
    src = jnp.concatenate(
        [ei[0].reshape(NW, EDGES_PER_TILE), pad_src], axis=1
    ).reshape(NW, NCHUNK, CHUNK)
    dst = jnp.concatenate(
        [ei[1].reshape(NW, EDGES_PER_TILE), pad_dst], axis=1
    ).reshape(NW, NCHUNK, CHUNK)
    partials = _scatter(src, dst, x)
    return _combine(partials, W, b.reshape(1, D))


# speedup vs baseline: 1.0373x; 1.0077x over previous
import functools

import jax
import jax.numpy as jnp
from jax import lax
from jax.experimental import pallas as pl
from jax.experimental.pallas import tpu as pltpu
from jax.experimental.pallas import tpu_sc as plsc

N_NODES = 10000
N_EDGES = 320000
D = 128

NC = 2
NS = 16
NW = NC * NS
CHUNK = 128
EDGES_PER_TILE = N_EDGES // NW
NCHUNK = 80
HALF = NCHUNK // 2
NPAIR_H = HALF // 2
ACC_ROWS = 10240
ROW_BLOCK = 2000


def _make_scatter_kernel():
    mesh = plsc.VectorSubcoreMesh(core_axis_name="c", subcore_axis_name="s")

    @functools.partial(
        pl.kernel,
        mesh=mesh,
        out_type=jax.ShapeDtypeStruct((NC, ACC_ROWS, D), jnp.float32),
        scratch_types=[
            pltpu.VMEM((HALF, CHUNK), jnp.int32),
            pltpu.VMEM((HALF, CHUNK), jnp.int32),
            pltpu.VMEM((CHUNK, D), jnp.float32),
            pltpu.VMEM((CHUNK, D), jnp.float32),
            pltpu.VMEM_SHARED((ACC_ROWS, D), jnp.float32),
            pltpu.SemaphoreType.DMA,
            pltpu.SemaphoreType.DMA,
        ],
    )
    def scatter_kernel(src_hbm, dst_hbm, x_hbm, out_hbm,
                       src_v, dst_v, rows0, rows1, acc, sem0a, sem1a):
        cid = lax.axis_index("c")
        sid = lax.axis_index("s")
        wid = sid * NC + cid

        idx_cp0 = pltpu.async_copy(
            src_hbm.at[wid, pl.ds(0, HALF)], src_v, sem0a)
        idx_cp1 = pltpu.async_copy(
            dst_hbm.at[wid, pl.ds(0, HALF)], dst_v, sem1a)

        zeros16 = jnp.zeros((16,), jnp.float32)

        def zrow(i, carry):
            for j in range(D // 16):
                rows0[i, pl.ds(j * 16, 16)] = zeros16
            return carry

        lax.fori_loop(0, CHUNK, zrow, 0)
        idx_cp0.wait()
        idx_cp1.wait()
        rows_per_tile = ACC_ROWS // NS

        def zacc(k, carry):
            pltpu.sync_copy(
                rows0, acc.at[pl.ds(sid * rows_per_tile + k * CHUNK, CHUNK)])
            return carry

        lax.fori_loop(0, rows_per_tile // CHUNK, zacc, 0)
        pltpu.async_copy(x_hbm.at[src_v.at[0]], rows0, sem0a)
        plsc.subcore_barrier()

        for h in range(NCHUNK // HALF):
            if h > 0:
                pltpu.sync_copy(src_hbm.at[wid, pl.ds(h * HALF, HALF)], src_v)
                pltpu.sync_copy(dst_hbm.at[wid, pl.ds(h * HALF, HALF)], dst_v)
                pltpu.async_copy(x_hbm.at[src_v.at[0]], rows0, sem0a)

            def step(k, carry):
                j0 = 2 * k
                j1 = j0 + 1
                pltpu.make_async_copy(
                    x_hbm.at[src_v.at[j0]], rows0, sem0a).wait()
                pltpu.async_copy(x_hbm.at[src_v.at[j1]], rows1, sem1a)
                pltpu.sync_copy(rows0, acc.at[dst_v.at[j0]], add=True)

                @pl.when(k < NPAIR_H - 1)
                def _():
                    pltpu.async_copy(x_hbm.at[src_v.at[j0 + 2]], rows0, sem0a)

                pltpu.make_async_copy(
                    x_hbm.at[src_v.at[j1]], rows1, sem1a).wait()
                pltpu.sync_copy(rows1, acc.at[dst_v.at[j1]], add=True)
                return carry

            lax.fori_loop(0, NPAIR_H, step, 0)
        plsc.subcore_barrier()

        pltpu.sync_copy(
            acc.at[pl.ds(sid * rows_per_tile, rows_per_tile)],
            out_hbm.at[cid, pl.ds(sid * rows_per_tile, rows_per_tile)])

    return scatter_kernel


_scatter = _make_scatter_kernel()


def _combine_body(p_ref, w_ref, b_ref, o_ref):
    agg = p_ref[0] + p_ref[1]
    o_ref[...] = (
        jnp.dot(agg, w_ref[...], preferred_element_type=jnp.float32)
        + b_ref[...])


_combine = pl.pallas_call(
    _combine_body,
    grid=(N_NODES // ROW_BLOCK,),
    in_specs=[
        pl.BlockSpec((NC, ROW_BLOCK, D), lambda i: (0, i, 0)),
        pl.BlockSpec((D, D), lambda i: (0, 0)),
        pl.BlockSpec((1, D), lambda i: (0, 0)),
    ],
    out_specs=pl.BlockSpec((ROW_BLOCK, D), lambda i: (i, 0)),
    out_shape=jax.ShapeDtypeStruct((N_NODES, D), jnp.float32),
)


def kernel(x, edge_index, W, b):
    ei = edge_index.astype(jnp.int32)
    ppt = NCHUNK * CHUNK - EDGES_PER_TILE
    lane = jnp.arange(ppt, dtype=jnp.int32)[None, :]
    tile = jnp.arange(NW, dtype=jnp.int32)[:, None]
    pad_src = (tile * 113 + lane * 89) % N_NODES
    pad_dst = N_NODES + (tile * 7 + lane) % (ACC_ROWS - N_NODES)
    src = jnp.concatenate(
        [ei[0].reshape(NW, EDGES_PER_TILE), pad_src], axis=1
    ).reshape(NW, NCHUNK, CHUNK)
    dst = jnp.concatenate(
        [ei[1].reshape(NW, EDGES_PER_TILE), pad_dst], axis=1
    ).reshape(NW, NCHUNK, CHUNK)
    partials = _scatter(src, dst, x)
    return _combine(partials, W, b.reshape(1, D))
